# Initial kernel scaffold; baseline (speedup 1.0000x reference)
import jax
import jax.numpy as jnp
from jax.experimental import pallas as pl


def kernel(x, edge_index, edge_weight, W0, b0, gamma0, beta0, W1, b1):
    raise NotImplementedError("write your pallas kernel here")


# speedup vs baseline: 4.1198x; 4.1198x over previous
import functools

import jax
import jax.numpy as jnp
from jax import lax
from jax.experimental import pallas as pl
from jax.experimental.pallas import tpu as pltpu
from jax.experimental.pallas import tpu_sc as plsc

D = 128
LANES = 16
NC = 2
NS = 16
NT = NC * NS
E_CHUNK = 128


def _linear_body(x_ref, w_ref, b_ref, o_ref):
    o_ref[...] = (
        jnp.dot(x_ref[...], w_ref[...], preferred_element_type=jnp.float32)
        + b_ref[...]
    )


def _tc_linear(x, wt, b):
    n = x.shape[0]
    return pl.pallas_call(
        _linear_body,
        out_shape=jax.ShapeDtypeStruct((n, D), jnp.float32),
    )(x, wt, b)


def _mid_body(p0_ref, p1_ref, g_ref, be_ref, w_ref, b_ref, o_ref):
    h = p0_ref[...] + p1_ref[...]
    mean = jnp.mean(h, axis=0, keepdims=True)
    d = h - mean
    var = jnp.mean(d * d, axis=0, keepdims=True)
    hn = d * lax.rsqrt(var + 1e-5) * g_ref[...] + be_ref[...]
    hn = jnp.maximum(hn, 0.0)
    o_ref[...] = (
        jnp.dot(hn, w_ref[...], preferred_element_type=jnp.float32) + b_ref[...]
    )


def _tc_mid(p0, p1, gamma, beta, wt, b):
    n = p0.shape[0]
    return pl.pallas_call(
        _mid_body,
        out_shape=jax.ShapeDtypeStruct((n, D), jnp.float32),
    )(p0, p1, gamma, beta, wt, b)


def _combine_body(p0_ref, p1_ref, o_ref):
    o_ref[...] = p0_ref[...] + p1_ref[...]


def _tc_combine(p0, p1):
    n = p0.shape[0]
    return pl.pallas_call(
        _combine_body,
        out_shape=jax.ShapeDtypeStruct((n, D), jnp.float32),
    )(p0, p1)


@functools.cache
def _make_spmm(n_nodes, n_chunks):
    n_pad = -(-n_nodes // (NS * E_CHUNK)) * (NS * E_CHUNK)
    rows_per_tile = n_pad // NS
    z_rows = E_CHUNK
    n_zcopies = rows_per_tile // z_rows

    mesh = plsc.VectorSubcoreMesh(core_axis_name="c", subcore_axis_name="s")

    @functools.partial(
        pl.kernel,
        out_type=jax.ShapeDtypeStruct((NC, n_pad, D), jnp.float32),
        mesh=mesh,
        scratch_types=[
            pltpu.VMEM((n_chunks, E_CHUNK), jnp.int32),
            pltpu.VMEM((n_chunks, E_CHUNK), jnp.int32),
            pltpu.VMEM((n_chunks * E_CHUNK,), jnp.float32),
            pltpu.VMEM((E_CHUNK, D), jnp.float32),
            pltpu.VMEM_SHARED((n_pad, D), jnp.float32),
            pltpu.SemaphoreType.DMA,
        ],
    )
    def spmm(h_hbm, row_hbm, col_hbm, w_hbm, out_hbm,
             col_v, row_v, w_v, rows_v, acc, sem):
        cid = lax.axis_index("c")
        sid = lax.axis_index("s")
        tid = cid * NS + sid

        pltpu.sync_copy(col_hbm.at[tid], col_v)
        pltpu.sync_copy(row_hbm.at[tid], row_v)
        pltpu.sync_copy(w_hbm.at[tid], w_v)

        zero = jnp.zeros((LANES,), jnp.float32)

        def _zrow(i, carry):
            for j in range(D // LANES):
                rows_v[i, pl.ds(j * LANES, LANES)] = zero
            return carry

        lax.fori_loop(0, E_CHUNK, _zrow, 0)
        for k in range(n_zcopies):
            pltpu.sync_copy(
                rows_v.at[pl.ds(0, z_rows)],
                acc.at[pl.ds(sid * rows_per_tile + k * z_rows, z_rows)],
            )
        plsc.subcore_barrier()

        def _chunk(j, carry):
            pltpu.async_copy(h_hbm.at[col_v.at[j]], rows_v, sem).wait()

            def _scale(g, c2):
                wv = w_v[pl.ds(j * E_CHUNK + g * LANES, LANES)]
                for l in range(LANES):
                    w = wv[l]
                    e = g * LANES + l
                    for f in range(D // LANES):
                        sl = pl.ds(f * LANES, LANES)
                        rows_v[e, sl] = rows_v[e, sl] * w
                return c2

            lax.fori_loop(0, E_CHUNK // LANES, _scale, 0)

            pltpu.sync_copy(rows_v, acc.at[row_v.at[j]], add=True)
            return carry

        lax.fori_loop(0, n_chunks, _chunk, 0)
        plsc.subcore_barrier()

        pltpu.sync_copy(
            acc.at[pl.ds(sid * rows_per_tile, rows_per_tile)],
            out_hbm.at[cid, pl.ds(sid * rows_per_tile, rows_per_tile)],
        )

    return spmm


def kernel(x, edge_index, edge_weight, W0, b0, gamma0, beta0, W1, b1):
    n = x.shape[0]
    e = edge_index.shape[1]

    n_chunks = -(-e // (NT * E_CHUNK))
    e_pad = NT * n_chunks * E_CHUNK
    pad = e_pad - e

    row = jnp.pad(edge_index[0].astype(jnp.int32), (0, pad))
    col = jnp.pad(edge_index[1].astype(jnp.int32), (0, pad))
    w = jnp.pad(edge_weight, (0, pad))
    row = row.reshape(NT, n_chunks, E_CHUNK)
    col = col.reshape(NT, n_chunks, E_CHUNK)
    w = w.reshape(NT, n_chunks * E_CHUNK)

    spmm = _make_spmm(n, n_chunks)

    h0 = _tc_linear(x, W0.T, b0.reshape(1, D))
    p = spmm(h0, row, col, w)
    h2 = _tc_mid(p[0, :n], p[1, :n], gamma0.reshape(1, D), beta0.reshape(1, D),
                 W1.T, b1.reshape(1, D))
    p2 = spmm(h2, row, col, w)
    return _tc_combine(p2[0, :n], p2[1, :n])
